# Initial kernel scaffold; baseline (speedup 1.0000x reference)
import jax
import jax.numpy as jnp
from jax.experimental import pallas as pl


def kernel(node_scalars, edge_feats, d, src_idxs, dst_idxs, W1, b1, W2, b2, ln_gamma, ln_beta):
    raise NotImplementedError("write your pallas kernel here")


# speedup vs baseline: 1.5129x; 1.5129x over previous
import functools

import jax
import jax.numpy as jnp
from jax import lax
from jax.experimental import pallas as pl
from jax.experimental.pallas import tpu as pltpu
from jax.experimental.pallas import tpu_sc as plsc

N_NODES = 50000
N_EDGES = 800000
NF = 64
RBF = 32

_NC = 2
_NS = 16
_NW = _NC * _NS

_EPW = N_EDGES // _NW
_CHUNK = 1000
_NCHUNKS = _EPW // _CHUNK


def _node_proj_body(ns_ref, w_ref, out_ref):
    out_ref[...] = jnp.dot(
        ns_ref[...], w_ref[...],
        preferred_element_type=jnp.float32,
        precision=lax.Precision.HIGHEST,
    )


def _node_proj(ns, w_sd):
    blk = 2000
    return pl.pallas_call(
        _node_proj_body,
        grid=(N_NODES // blk,),
        in_specs=[
            pl.BlockSpec((blk, NF), lambda i: (i, 0)),
            pl.BlockSpec((NF, 2 * NF), lambda i: (0, 0)),
        ],
        out_specs=pl.BlockSpec((blk, 2 * NF), lambda i: (i, 0)),
        out_shape=jax.ShapeDtypeStruct((N_NODES, 2 * NF), jnp.float32),
    )(ns, w_sd)


def _gather_add_body(ps_hbm, pd_hbm, src_hbm, dst_hbm, out_hbm,
                     sidx, didx, bufa, bufb, sema, semb):
    wid = lax.axis_index("s") * _NC + lax.axis_index("c")
    base = wid * _EPW

    def chunk(k, carry):
        start = base + k * _CHUNK
        pltpu.sync_copy(src_hbm.at[pl.ds(start, _CHUNK)], sidx)
        pltpu.sync_copy(dst_hbm.at[pl.ds(start, _CHUNK)], didx)
        ca = pltpu.async_copy(ps_hbm.at[sidx], bufa, sema)
        cb = pltpu.async_copy(pd_hbm.at[didx], bufb, semb)
        ca.wait()
        cb.wait()

        def add_row(i, c2):
            for j in range(4):
                sl = pl.ds(j * 16, 16)
                bufa[i, sl] = bufa[i, sl] + bufb[i, sl]
            return c2

        lax.fori_loop(0, _CHUNK, add_row, 0)
        pltpu.sync_copy(bufa, out_hbm.at[pl.ds(start, _CHUNK)])
        return carry

    lax.fori_loop(0, _NCHUNKS, chunk, 0)


def _gather_add(ps, pd, src, dst):
    mesh = plsc.VectorSubcoreMesh(core_axis_name="c", subcore_axis_name="s")
    fn = pl.kernel(
        _gather_add_body,
        mesh=mesh,
        compiler_params=pltpu.CompilerParams(use_tc_tiling_on_sc=False),
        out_type=jax.ShapeDtypeStruct((N_EDGES, NF), jnp.float32),
        scratch_types=[
            pltpu.VMEM((_CHUNK,), jnp.int32),
            pltpu.VMEM((_CHUNK,), jnp.int32),
            pltpu.VMEM((_CHUNK, NF), jnp.float32),
            pltpu.VMEM((_CHUNK, NF), jnp.float32),
            pltpu.SemaphoreType.DMA,
            pltpu.SemaphoreType.DMA,
        ],
    )
    return fn(ps, pd, src, dst)


def _edge_mlp_body(g_ref, ef_ref, d_ref, w1e_ref, w1r_ref, b1_ref,
                   w2_ref, b2_ref, gam_ref, bet_ref, out_ref):
    ef = ef_ref[...]
    h = (
        g_ref[...]
        + jnp.dot(ef, w1e_ref[...], preferred_element_type=jnp.float32,
                  precision=lax.Precision.HIGHEST)
        + jnp.dot(d_ref[...], w1r_ref[...], preferred_element_type=jnp.float32,
                  precision=lax.Precision.HIGHEST)
        + b1_ref[...]
    )
    h = h * jax.nn.sigmoid(h)
    h = jnp.dot(h, w2_ref[...], preferred_element_type=jnp.float32,
                precision=lax.Precision.HIGHEST) + b2_ref[...]
    h = h * jax.nn.sigmoid(h)
    y = ef + h
    mean = jnp.mean(y, axis=1, keepdims=True)
    var = jnp.mean(jnp.square(y - mean), axis=1, keepdims=True)
    out_ref[...] = (y - mean) * lax.rsqrt(var + 1e-5) * gam_ref[...] + bet_ref[...]


def _edge_mlp(g, ef, d, w1e, w1r, b1, w2, b2, gamma, beta):
    blk = 4000
    cst = lambda i: (0, 0)
    return pl.pallas_call(
        _edge_mlp_body,
        grid=(N_EDGES // blk,),
        in_specs=[
            pl.BlockSpec((blk, NF), lambda i: (i, 0)),
            pl.BlockSpec((blk, NF), lambda i: (i, 0)),
            pl.BlockSpec((blk, RBF), lambda i: (i, 0)),
            pl.BlockSpec((NF, NF), cst),
            pl.BlockSpec((RBF, NF), cst),
            pl.BlockSpec((1, NF), cst),
            pl.BlockSpec((NF, NF), cst),
            pl.BlockSpec((1, NF), cst),
            pl.BlockSpec((1, NF), cst),
            pl.BlockSpec((1, NF), cst),
        ],
        out_specs=pl.BlockSpec((blk, NF), lambda i: (i, 0)),
        out_shape=jax.ShapeDtypeStruct((N_EDGES, NF), jnp.float32),
    )(g, ef, d, w1e, w1r, b1, w2, b2, gamma, beta)


def kernel(node_scalars, edge_feats, d, src_idxs, dst_idxs,
           W1, b1, W2, b2, ln_gamma, ln_beta):
    w_sd = jnp.concatenate([W1[:NF], W1[NF:2 * NF]], axis=1)
    w1e = W1[2 * NF:3 * NF]
    w1r = W1[3 * NF:]

    p = _node_proj(node_scalars, w_sd)
    g = _gather_add(p[:, :NF], p[:, NF:],
                    src_idxs.astype(jnp.int32), dst_idxs.astype(jnp.int32))
    return _edge_mlp(
        g, edge_feats, d, w1e, w1r,
        b1.reshape(1, NF), W2, b2.reshape(1, NF),
        ln_gamma.reshape(1, NF), ln_beta.reshape(1, NF),
    )


# speedup vs baseline: 2.4373x; 1.6110x over previous
import functools

import jax
import jax.numpy as jnp
from jax import lax
from jax.experimental import pallas as pl
from jax.experimental.pallas import tpu as pltpu
from jax.experimental.pallas import tpu_sc as plsc

N_NODES = 50000
N_EDGES = 800000
NF = 64
RBF = 32

_NC = 2
_NS = 16
_NW = _NC * _NS

_EPW = N_EDGES // _NW
_CHUNK = 1000
_NCHUNKS = _EPW // _CHUNK


def _node_proj_body(ns_ref, w_ref, out_ref):
    out_ref[...] = jnp.dot(
        ns_ref[...], w_ref[...],
        preferred_element_type=jnp.float32,
        precision=lax.Precision.HIGHEST,
    )


def _node_proj(ns, w_sd):
    blk = 2000
    return pl.pallas_call(
        _node_proj_body,
        grid=(N_NODES // blk,),
        in_specs=[
            pl.BlockSpec((blk, NF), lambda i: (i, 0)),
            pl.BlockSpec((NF, 2 * NF), lambda i: (0, 0)),
        ],
        out_specs=pl.BlockSpec((blk, 2 * NF), lambda i: (i, 0)),
        out_shape=jax.ShapeDtypeStruct((N_NODES, 2 * NF), jnp.float32),
    )(ns, w_sd)


def _gather_add_body(ps_hbm, pd_hbm, src_hbm, dst_hbm, out_hbm,
                     sidx, didx, bufa, bufb, sema, semb):
    wid = lax.axis_index("s") * _NC + lax.axis_index("c")
    base = wid * _EPW

    def chunk(k, carry):
        start = base + k * _CHUNK
        pltpu.sync_copy(src_hbm.at[pl.ds(start, _CHUNK)], sidx)
        pltpu.sync_copy(dst_hbm.at[pl.ds(start, _CHUNK)], didx)
        ca = pltpu.async_copy(ps_hbm.at[sidx], bufa, sema)
        cb = pltpu.async_copy(pd_hbm.at[didx], bufb, semb)
        ca.wait()
        cb.wait()

        def add_row(i, c2):
            for j in range(4):
                sl = pl.ds(j * 16, 16)
                bufa[i, sl] = bufa[i, sl] + bufb[i, sl]
            return c2

        lax.fori_loop(0, _CHUNK, add_row, 0)
        pltpu.sync_copy(bufa, out_hbm.at[pl.ds(start, _CHUNK)])
        return carry

    lax.fori_loop(0, _NCHUNKS, chunk, 0)


def _gather_add(ps, pd, src, dst):
    mesh = plsc.VectorSubcoreMesh(core_axis_name="c", subcore_axis_name="s")
    fn = pl.kernel(
        _gather_add_body,
        mesh=mesh,
        compiler_params=pltpu.CompilerParams(use_tc_tiling_on_sc=False),
        out_type=jax.ShapeDtypeStruct((N_EDGES, NF), jnp.float32),
        scratch_types=[
            pltpu.VMEM((_CHUNK,), jnp.int32),
            pltpu.VMEM((_CHUNK,), jnp.int32),
            pltpu.VMEM((_CHUNK, NF), jnp.float32),
            pltpu.VMEM((_CHUNK, NF), jnp.float32),
            pltpu.SemaphoreType.DMA,
            pltpu.SemaphoreType.DMA,
        ],
    )
    return fn(ps, pd, src, dst)


def _edge_mlp_body(g_ref, ef_ref, d_ref, w1e_ref, w1r_ref, b1_ref,
                   w2_ref, b2_ref, gam_ref, bet_ref, out_ref):
    ef = ef_ref[...]
    h = (
        g_ref[...]
        + jnp.dot(ef, w1e_ref[...], preferred_element_type=jnp.float32,
                  precision=lax.Precision.DEFAULT)
        + jnp.dot(d_ref[...], w1r_ref[...], preferred_element_type=jnp.float32,
                  precision=lax.Precision.DEFAULT)
        + b1_ref[...]
    )
    h = h * jax.nn.sigmoid(h)
    h = jnp.dot(h, w2_ref[...], preferred_element_type=jnp.float32,
                precision=lax.Precision.DEFAULT) + b2_ref[...]
    h = h * jax.nn.sigmoid(h)
    y = ef + h
    mean = jnp.mean(y, axis=1, keepdims=True)
    var = jnp.mean(jnp.square(y - mean), axis=1, keepdims=True)
    out_ref[...] = (y - mean) * lax.rsqrt(var + 1e-5) * gam_ref[...] + bet_ref[...]


def _edge_mlp(g, ef, d, w1e, w1r, b1, w2, b2, gamma, beta):
    blk = 4000
    cst = lambda i: (0, 0)
    return pl.pallas_call(
        _edge_mlp_body,
        grid=(N_EDGES // blk,),
        in_specs=[
            pl.BlockSpec((blk, NF), lambda i: (i, 0)),
            pl.BlockSpec((blk, NF), lambda i: (i, 0)),
            pl.BlockSpec((blk, RBF), lambda i: (i, 0)),
            pl.BlockSpec((NF, NF), cst),
            pl.BlockSpec((RBF, NF), cst),
            pl.BlockSpec((1, NF), cst),
            pl.BlockSpec((NF, NF), cst),
            pl.BlockSpec((1, NF), cst),
            pl.BlockSpec((1, NF), cst),
            pl.BlockSpec((1, NF), cst),
        ],
        out_specs=pl.BlockSpec((blk, NF), lambda i: (i, 0)),
        out_shape=jax.ShapeDtypeStruct((N_EDGES, NF), jnp.float32),
    )(g, ef, d, w1e, w1r, b1, w2, b2, gamma, beta)


def kernel(node_scalars, edge_feats, d, src_idxs, dst_idxs,
           W1, b1, W2, b2, ln_gamma, ln_beta):
    w_sd = jnp.concatenate([W1[:NF], W1[NF:2 * NF]], axis=1)
    w1e = W1[2 * NF:3 * NF]
    w1r = W1[3 * NF:]

    p = _node_proj(node_scalars, w_sd)
    g = _gather_add(p[:, :NF], p[:, NF:],
                    src_idxs.astype(jnp.int32), dst_idxs.astype(jnp.int32))
    return _edge_mlp(
        g, edge_feats, d, w1e, w1r,
        b1.reshape(1, NF), W2, b2.reshape(1, NF),
        ln_gamma.reshape(1, NF), ln_beta.reshape(1, NF),
    )


# speedup vs baseline: 4.0347x; 1.6554x over previous
import functools

import jax
import jax.numpy as jnp
from jax import lax
from jax.experimental import pallas as pl
from jax.experimental.pallas import tpu as pltpu
from jax.experimental.pallas import tpu_sc as plsc

N_NODES = 50000
N_EDGES = 800000
NF = 64
RBF = 32

_NC = 2
_NS = 16
_NW = _NC * _NS

_EPW = N_EDGES // _NW
_CHUNK = 1000
_NCHUNKS = _EPW // _CHUNK


def _node_proj_body(ns_ref, w_ref, out_ref):
    out_ref[...] = jnp.dot(
        ns_ref[...], w_ref[...],
        preferred_element_type=jnp.float32,
        precision=lax.Precision.HIGHEST,
    )


def _node_proj(ns, w_sd):
    blk = 2000
    return pl.pallas_call(
        _node_proj_body,
        grid=(N_NODES // blk,),
        in_specs=[
            pl.BlockSpec((blk, NF), lambda i: (i, 0)),
            pl.BlockSpec((NF, 2 * NF), lambda i: (0, 0)),
        ],
        out_specs=pl.BlockSpec((blk, 2 * NF), lambda i: (i, 0)),
        out_shape=jax.ShapeDtypeStruct((N_NODES, 2 * NF), jnp.float32),
    )(ns, w_sd)


def _gather_add_body(ps_hbm, pd_hbm, src_hbm, dst_hbm, out_hbm,
                     sidx, didx, bufa, bufb, sema, semb):
    wid = lax.axis_index("s") * _NC + lax.axis_index("c")
    base = wid * _EPW

    def chunk(k, carry):
        start = base + k * _CHUNK
        pltpu.sync_copy(src_hbm.at[pl.ds(start, _CHUNK)], sidx)
        pltpu.sync_copy(dst_hbm.at[pl.ds(start, _CHUNK)], didx)
        ca = pltpu.async_copy(ps_hbm.at[sidx], bufa, sema)
        cb = pltpu.async_copy(pd_hbm.at[didx], bufb, semb)
        ca.wait()
        cb.wait()

        def add_row(i, c2):
            for j in range(4):
                sl = pl.ds(j * 16, 16)
                bufa[i, sl] = bufa[i, sl] + bufb[i, sl]
            return c2

        lax.fori_loop(0, _CHUNK, add_row, 0)
        pltpu.sync_copy(bufa, out_hbm.at[pl.ds(start, _CHUNK)])
        return carry

    lax.fori_loop(0, _NCHUNKS, chunk, 0)


def _gather_add(ps, pd, src, dst):
    mesh = plsc.VectorSubcoreMesh(core_axis_name="c", subcore_axis_name="s")
    fn = pl.kernel(
        _gather_add_body,
        mesh=mesh,
        compiler_params=pltpu.CompilerParams(use_tc_tiling_on_sc=False),
        out_type=jax.ShapeDtypeStruct((N_EDGES, NF), jnp.float32),
        scratch_types=[
            pltpu.VMEM((_CHUNK,), jnp.int32),
            pltpu.VMEM((_CHUNK,), jnp.int32),
            pltpu.VMEM((_CHUNK, NF), jnp.float32),
            pltpu.VMEM((_CHUNK, NF), jnp.float32),
            pltpu.SemaphoreType.DMA,
            pltpu.SemaphoreType.DMA,
        ],
    )
    return fn(ps, pd, src, dst)


def _edge_mlp_t_body(g_ref, ef_ref, d_ref, w1e_t_ref, w1r_t_ref, b1_ref,
                     w2_t_ref, b2_ref, gam_ref, bet_ref, out_ref):
    ef = ef_ref[...]
    gt = jnp.transpose(g_ref[...])
    h = (
        gt
        + jnp.dot(w1e_t_ref[...], ef, preferred_element_type=jnp.float32,
                  precision=lax.Precision.DEFAULT)
        + jnp.dot(w1r_t_ref[...], d_ref[...], preferred_element_type=jnp.float32,
                  precision=lax.Precision.DEFAULT)
        + b1_ref[...]
    )
    h = h * jax.nn.sigmoid(h)
    h = jnp.dot(w2_t_ref[...], h, preferred_element_type=jnp.float32,
                precision=lax.Precision.DEFAULT) + b2_ref[...]
    h = h * jax.nn.sigmoid(h)
    y = ef + h
    mean = jnp.mean(y, axis=0, keepdims=True)
    var = jnp.mean(jnp.square(y - mean), axis=0, keepdims=True)
    out_ref[...] = (y - mean) * lax.rsqrt(var + 1e-5) * gam_ref[...] + bet_ref[...]


def _edge_mlp_t(g, ef_t, d_t, w1e_t, w1r_t, b1c, w2_t, b2c, gam_c, bet_c):
    blk = 6400
    cst = lambda i: (0, 0)
    return pl.pallas_call(
        _edge_mlp_t_body,
        grid=(N_EDGES // blk,),
        in_specs=[
            pl.BlockSpec((blk, NF), lambda i: (i, 0)),
            pl.BlockSpec((NF, blk), lambda i: (0, i)),
            pl.BlockSpec((RBF, blk), lambda i: (0, i)),
            pl.BlockSpec((NF, NF), cst),
            pl.BlockSpec((NF, RBF), cst),
            pl.BlockSpec((NF, 1), cst),
            pl.BlockSpec((NF, NF), cst),
            pl.BlockSpec((NF, 1), cst),
            pl.BlockSpec((NF, 1), cst),
            pl.BlockSpec((NF, 1), cst),
        ],
        out_specs=pl.BlockSpec((NF, blk), lambda i: (0, i)),
        out_shape=jax.ShapeDtypeStruct((NF, N_EDGES), jnp.float32),
    )(g, ef_t, d_t, w1e_t, w1r_t, b1c, w2_t, b2c, gam_c, bet_c)


def kernel(node_scalars, edge_feats, d, src_idxs, dst_idxs,
           W1, b1, W2, b2, ln_gamma, ln_beta):
    w_sd = jnp.concatenate([W1[:NF], W1[NF:2 * NF]], axis=1)
    w1e_t = W1[2 * NF:3 * NF].T
    w1r_t = W1[3 * NF:].T

    p = _node_proj(node_scalars, w_sd)
    g = _gather_add(p[:, :NF], p[:, NF:],
                    src_idxs.astype(jnp.int32), dst_idxs.astype(jnp.int32))
    out_t = _edge_mlp_t(
        g, edge_feats.T, d.T, w1e_t, w1r_t,
        b1.reshape(NF, 1), W2.T, b2.reshape(NF, 1),
        ln_gamma.reshape(NF, 1), ln_beta.reshape(NF, 1),
    )
    return out_t.T


# speedup vs baseline: 4.0726x; 1.0094x over previous
import functools

import jax
import jax.numpy as jnp
from jax import lax
from jax.experimental import pallas as pl
from jax.experimental.pallas import tpu as pltpu
from jax.experimental.pallas import tpu_sc as plsc

N_NODES = 50000
N_EDGES = 800000
NF = 64
RBF = 32

_NC = 2
_NS = 16
_NW = _NC * _NS

_EPW = N_EDGES // _NW
_CHUNK = 1000
_NCHUNKS = _EPW // _CHUNK

_BLK = 6400
_NBLK = N_EDGES // _BLK


def _node_proj_body(ns_t_ref, w_ref, p_ref):
    p_ref[...] = lax.dot_general(
        ns_t_ref[...], w_ref[...],
        dimension_numbers=(((0,), (0,)), ((), ())),
        preferred_element_type=jnp.float32,
        precision=lax.Precision.DEFAULT,
    )


def _node_proj(ns_t, w_sd):
    return pl.pallas_call(
        _node_proj_body,
        out_shape=jax.ShapeDtypeStruct((N_NODES, 2 * NF), jnp.float32),
    )(ns_t, w_sd)


def _gather_add_body(tab_hbm, src_hbm, dst_hbm, out_hbm,
                     sidx, didx, bufa, bufb, sema, semb):
    wid = lax.axis_index("s") * _NC + lax.axis_index("c")
    base = wid * _EPW

    def chunk(k, carry):
        start = base + k * _CHUNK
        pltpu.sync_copy(src_hbm.at[pl.ds(start, _CHUNK)], sidx)
        pltpu.sync_copy(dst_hbm.at[pl.ds(start, _CHUNK)], didx)
        ca = pltpu.async_copy(tab_hbm.at[sidx], bufa, sema)
        cb = pltpu.async_copy(tab_hbm.at[didx], bufb, semb)
        ca.wait()
        cb.wait()

        def add_row(i, c2):
            for j in range(4):
                sl = pl.ds(j * 16, 16)
                bufa[i, sl] = bufa[i, sl] + bufb[i, sl]
            return c2

        lax.fori_loop(0, _CHUNK, add_row, 0)
        pltpu.sync_copy(bufa, out_hbm.at[pl.ds(start, _CHUNK)])
        return carry

    lax.fori_loop(0, _NCHUNKS, chunk, 0)


def _gather_add(table, src2, dst2):
    mesh = plsc.VectorSubcoreMesh(core_axis_name="c", subcore_axis_name="s")
    fn = pl.kernel(
        _gather_add_body,
        mesh=mesh,
        compiler_params=pltpu.CompilerParams(use_tc_tiling_on_sc=False),
        out_type=jax.ShapeDtypeStruct((N_EDGES, NF), jnp.float32),
        scratch_types=[
            pltpu.VMEM((_CHUNK,), jnp.int32),
            pltpu.VMEM((_CHUNK,), jnp.int32),
            pltpu.VMEM((_CHUNK, NF), jnp.float32),
            pltpu.VMEM((_CHUNK, NF), jnp.float32),
            pltpu.SemaphoreType.DMA,
            pltpu.SemaphoreType.DMA,
        ],
    )
    return fn(table, src2, dst2)


def _edge_mlp_t_body(g_ref, ef_ref, d_ref, w1e_t_ref, w1r_t_ref, b1_ref,
                     w2_t_ref, b2_ref, gam_ref, bet_ref, out_ref):
    ef = ef_ref[...]
    gt = jnp.transpose(g_ref[...])
    g_t = jnp.concatenate([gt[:NF, :], gt[NF:, :]], axis=1)
    h = (
        g_t
        + jnp.dot(w1e_t_ref[...], ef, preferred_element_type=jnp.float32,
                  precision=lax.Precision.DEFAULT)
        + jnp.dot(w1r_t_ref[...], d_ref[...], preferred_element_type=jnp.float32,
                  precision=lax.Precision.DEFAULT)
        + b1_ref[...]
    )
    h = h * jax.nn.sigmoid(h)
    h = jnp.dot(w2_t_ref[...], h, preferred_element_type=jnp.float32,
                precision=lax.Precision.DEFAULT) + b2_ref[...]
    h = h * jax.nn.sigmoid(h)
    y = ef + h
    mean = jnp.mean(y, axis=0, keepdims=True)
    var = jnp.mean(jnp.square(y - mean), axis=0, keepdims=True)
    out_ref[...] = (y - mean) * lax.rsqrt(var + 1e-5) * gam_ref[...] + bet_ref[...]


def _edge_mlp_t(g128, ef_t, d_t, w1e_t, w1r_t, b1c, w2_t, b2c, gam_c, bet_c):
    cst = lambda i: (0, 0)
    return pl.pallas_call(
        _edge_mlp_t_body,
        grid=(_NBLK,),
        in_specs=[
            pl.BlockSpec((_BLK // 2, 2 * NF), lambda i: (i, 0)),
            pl.BlockSpec((NF, _BLK), lambda i: (0, i)),
            pl.BlockSpec((RBF, _BLK), lambda i: (0, i)),
            pl.BlockSpec((NF, NF), cst),
            pl.BlockSpec((NF, RBF), cst),
            pl.BlockSpec((NF, 1), cst),
            pl.BlockSpec((NF, NF), cst),
            pl.BlockSpec((NF, 1), cst),
            pl.BlockSpec((NF, 1), cst),
            pl.BlockSpec((NF, 1), cst),
        ],
        out_specs=pl.BlockSpec((NF, _BLK), lambda i: (0, i)),
        out_shape=jax.ShapeDtypeStruct((NF, N_EDGES), jnp.float32),
    )(g128, ef_t, d_t, w1e_t, w1r_t, b1c, w2_t, b2c, gam_c, bet_c)


def _pair_interleave(x):
    return x.reshape(_NBLK, 2, _BLK // 2).transpose(0, 2, 1).reshape(-1)


def kernel(node_scalars, edge_feats, d, src_idxs, dst_idxs,
           W1, b1, W2, b2, ln_gamma, ln_beta):
    w_sd = jnp.concatenate([W1[:NF], W1[NF:2 * NF]], axis=1)
    w1e_t = W1[2 * NF:3 * NF].T
    w1r_t = W1[3 * NF:].T

    src2 = _pair_interleave(src_idxs.astype(jnp.int32) * 2)
    dst2 = _pair_interleave(dst_idxs.astype(jnp.int32) * 2 + 1)

    p = _node_proj(node_scalars.T, w_sd)
    table = p.reshape(2 * N_NODES, NF)
    g = _gather_add(table, src2, dst2)
    g128 = g.reshape(N_EDGES // 2, 2 * NF)
    out_t = _edge_mlp_t(
        g128, edge_feats.T, d.T, w1e_t, w1r_t,
        b1.reshape(NF, 1), W2.T, b2.reshape(NF, 1),
        ln_gamma.reshape(NF, 1), ln_beta.reshape(NF, 1),
    )
    return out_t.T


# speedup vs baseline: 6.2220x; 1.5278x over previous
import functools

import jax
import jax.numpy as jnp
from jax import lax
from jax.experimental import pallas as pl
from jax.experimental.pallas import tpu as pltpu
from jax.experimental.pallas import tpu_sc as plsc

N_NODES = 50000
N_EDGES = 800000
NF = 64
RBF = 32

_NC = 2
_NS = 16
_NW = _NC * _NS

_BLK = 6400
_NBLK = N_EDGES // _BLK

_SUB = 400
_PPB = _BLK // 2
_SPB = _PPB // _SUB
_JOBS = (N_EDGES // 2) // _SUB


def _node_proj_body(ns_t_ref, w_ref, p_ref):
    p_ref[...] = lax.dot_general(
        ns_t_ref[...], w_ref[...],
        dimension_numbers=(((0,), (0,)), ((), ())),
        preferred_element_type=jnp.float32,
        precision=lax.Precision.DEFAULT,
    )


def _node_proj(ns_t, w_sd):
    return pl.pallas_call(
        _node_proj_body,
        out_shape=jax.ShapeDtypeStruct((N_NODES, 2 * NF), jnp.float32),
    )(ns_t, w_sd)


def _gather_add_body(tab_hbm, src_hbm, dst_hbm, out_hbm,
                     silo, dilo, sihi, dihi, bla, blb, bha, bhb,
                     s1, s2, s3, s4):
    wid = lax.axis_index("s") * _NC + lax.axis_index("c")
    njobs = (_JOBS - wid + _NW - 1) // _NW

    def job(k, carry):
        j = wid + k * _NW
        b = j // _SPB
        s = j % _SPB
        lo = b * _BLK + s * _SUB
        hi = lo + _PPB
        r0 = b * _PPB + s * _SUB
        pltpu.sync_copy(src_hbm.at[pl.ds(lo, _SUB)], silo)
        pltpu.sync_copy(dst_hbm.at[pl.ds(lo, _SUB)], dilo)
        pltpu.sync_copy(src_hbm.at[pl.ds(hi, _SUB)], sihi)
        pltpu.sync_copy(dst_hbm.at[pl.ds(hi, _SUB)], dihi)
        c1 = pltpu.async_copy(tab_hbm.at[silo], bla, s1)
        c2 = pltpu.async_copy(tab_hbm.at[dilo], blb, s2)
        c3 = pltpu.async_copy(tab_hbm.at[sihi], bha, s3)
        c4 = pltpu.async_copy(tab_hbm.at[dihi], bhb, s4)
        c1.wait()
        c2.wait()
        c3.wait()
        c4.wait()

        def add_row(i, c2_):
            for jj in range(4):
                sl = pl.ds(jj * 16, 16)
                bla[i, sl] = bla[i, sl] + blb[i, sl]
                bha[i, sl] = bha[i, sl] + bhb[i, sl]
            return c2_

        lax.fori_loop(0, _SUB, add_row, 0)
        pltpu.sync_copy(bla, out_hbm.at[pl.ds(r0, _SUB), pl.ds(0, NF)])
        pltpu.sync_copy(bha, out_hbm.at[pl.ds(r0, _SUB), pl.ds(NF, NF)])
        return carry

    lax.fori_loop(0, njobs, job, 0)


def _gather_add(table, src2, dst2):
    mesh = plsc.VectorSubcoreMesh(core_axis_name="c", subcore_axis_name="s")
    fn = pl.kernel(
        _gather_add_body,
        mesh=mesh,
        compiler_params=pltpu.CompilerParams(use_tc_tiling_on_sc=False),
        out_type=jax.ShapeDtypeStruct((N_EDGES // 2, 2 * NF), jnp.float32),
        scratch_types=[
            pltpu.VMEM((_SUB,), jnp.int32),
            pltpu.VMEM((_SUB,), jnp.int32),
            pltpu.VMEM((_SUB,), jnp.int32),
            pltpu.VMEM((_SUB,), jnp.int32),
            pltpu.VMEM((_SUB, NF), jnp.float32),
            pltpu.VMEM((_SUB, NF), jnp.float32),
            pltpu.VMEM((_SUB, NF), jnp.float32),
            pltpu.VMEM((_SUB, NF), jnp.float32),
            pltpu.SemaphoreType.DMA,
            pltpu.SemaphoreType.DMA,
            pltpu.SemaphoreType.DMA,
            pltpu.SemaphoreType.DMA,
        ],
    )
    return fn(table, src2, dst2)


def _edge_mlp_t_body(g_ref, ef_ref, d_ref, w1e_t_ref, w1r_t_ref, b1_ref,
                     w2_t_ref, b2_ref, gam_ref, bet_ref, out_ref):
    ef = ef_ref[...]
    gt = jnp.transpose(g_ref[...])
    g_t = jnp.concatenate([gt[:NF, :], gt[NF:, :]], axis=1)
    h = (
        g_t
        + jnp.dot(w1e_t_ref[...], ef, preferred_element_type=jnp.float32,
                  precision=lax.Precision.DEFAULT)
        + jnp.dot(w1r_t_ref[...], d_ref[...], preferred_element_type=jnp.float32,
                  precision=lax.Precision.DEFAULT)
        + b1_ref[...]
    )
    h = h * jax.nn.sigmoid(h)
    h = jnp.dot(w2_t_ref[...], h, preferred_element_type=jnp.float32,
                precision=lax.Precision.DEFAULT) + b2_ref[...]
    h = h * jax.nn.sigmoid(h)
    y = ef + h
    mean = jnp.mean(y, axis=0, keepdims=True)
    var = jnp.mean(jnp.square(y - mean), axis=0, keepdims=True)
    out_ref[...] = (y - mean) * lax.rsqrt(var + 1e-5) * gam_ref[...] + bet_ref[...]


def _edge_mlp_t(g128, ef_t, d_t, w1e_t, w1r_t, b1c, w2_t, b2c, gam_c, bet_c):
    cst = lambda i: (0, 0)
    return pl.pallas_call(
        _edge_mlp_t_body,
        grid=(_NBLK,),
        in_specs=[
            pl.BlockSpec((_BLK // 2, 2 * NF), lambda i: (i, 0)),
            pl.BlockSpec((NF, _BLK), lambda i: (0, i)),
            pl.BlockSpec((RBF, _BLK), lambda i: (0, i)),
            pl.BlockSpec((NF, NF), cst),
            pl.BlockSpec((NF, RBF), cst),
            pl.BlockSpec((NF, 1), cst),
            pl.BlockSpec((NF, NF), cst),
            pl.BlockSpec((NF, 1), cst),
            pl.BlockSpec((NF, 1), cst),
            pl.BlockSpec((NF, 1), cst),
        ],
        out_specs=pl.BlockSpec((NF, _BLK), lambda i: (0, i)),
        out_shape=jax.ShapeDtypeStruct((NF, N_EDGES), jnp.float32),
    )(g128, ef_t, d_t, w1e_t, w1r_t, b1c, w2_t, b2c, gam_c, bet_c)


def kernel(node_scalars, edge_feats, d, src_idxs, dst_idxs,
           W1, b1, W2, b2, ln_gamma, ln_beta):
    w_sd = jnp.concatenate([W1[:NF], W1[NF:2 * NF]], axis=1)
    w1e_t = W1[2 * NF:3 * NF].T
    w1r_t = W1[3 * NF:].T

    src2 = src_idxs.astype(jnp.int32) * 2
    dst2 = dst_idxs.astype(jnp.int32) * 2 + 1

    p = _node_proj(node_scalars.T, w_sd)
    table = p.reshape(2 * N_NODES, NF)
    g128 = _gather_add(table, src2, dst2)
    out_t = _edge_mlp_t(
        g128, edge_feats.T, d.T, w1e_t, w1r_t,
        b1.reshape(NF, 1), W2.T, b2.reshape(NF, 1),
        ln_gamma.reshape(NF, 1), ln_beta.reshape(NF, 1),
    )
    return out_t.T
